# Initial kernel scaffold; baseline (speedup 1.0000x reference)
import jax
import jax.numpy as jnp
from jax.experimental import pallas as pl


def kernel(rna_f, protein_f, all_edges, W_sage_l, b_sage, W_sage_r, W1, b1, W2, b2, W3, b3):
    raise NotImplementedError("write your pallas kernel here")


# speedup vs baseline: 2.8597x; 2.8597x over previous
import functools

import jax
import jax.numpy as jnp
from jax import lax
from jax.experimental import pallas as pl
from jax.experimental.pallas import tpu as pltpu
from jax.experimental.pallas import tpu_sc as plsc

N_RNA = 5000
N_PROT = 5000
NUM_NODES = N_RNA + N_PROT
D = 128
HID = 128
N_ALL_EDGES = 320000
E_POS = N_ALL_EDGES // 2
E = 2 * E_POS

NC = 2
NS = 16
NW = NC * NS
EW = E // NW
CH = 80
NCHUNK = EW // CH

DUMP_TILES = 10
ROWS_PER_TILE = NUM_NODES // DUMP_TILES
ZR = 40
NZCHUNK = ROWS_PER_TILE // ZR

_vec_mesh = plsc.VectorSubcoreMesh(core_axis_name="c", subcore_axis_name="s")


@functools.partial(
    pl.kernel,
    out_type=jax.ShapeDtypeStruct((E, D), jnp.float32),
    mesh=_vec_mesh,
    scratch_types=[
        pltpu.VMEM((EW,), jnp.int32),
        pltpu.VMEM((EW,), jnp.int32),
        pltpu.VMEM((CH, D), jnp.float32),
        pltpu.VMEM((CH, D), jnp.float32),
        pltpu.VMEM((CH, D), jnp.float32),
        pltpu.SemaphoreType.DMA,
        pltpu.SemaphoreType.DMA,
    ],
)
def _edge_features(nfea, src, dst, h_out, src_v, dst_v, bufa, bufb, bufh,
                   sema, semb):
    wid = lax.axis_index("s") * NC + lax.axis_index("c")
    base = wid * EW
    pltpu.sync_copy(src.at[pl.ds(base, EW)], src_v)
    pltpu.sync_copy(dst.at[pl.ds(base, EW)], dst_v)

    def chunk(c, carry):
        off = c * CH
        cpa = pltpu.async_copy(nfea.at[src_v.at[pl.ds(off, CH)]], bufa, sema)
        cpb = pltpu.async_copy(nfea.at[dst_v.at[pl.ds(off, CH)]], bufb, semb)
        cpa.wait()
        cpb.wait()

        def row(r, carry2):
            for k in range(D // 16):
                sl = pl.ds(k * 16, 16)
                bufh[r, sl] = bufa[r, sl] * bufb[r, sl]
            return carry2

        lax.fori_loop(0, CH, row, 0, unroll=False)
        pltpu.sync_copy(bufh, h_out.at[pl.ds(base + off, CH)])
        return carry

    lax.fori_loop(0, NCHUNK, chunk, 0, unroll=False)


@functools.partial(
    pl.kernel,
    out_type=jax.ShapeDtypeStruct((NC, NUM_NODES, D), jnp.float32),
    mesh=_vec_mesh,
    scratch_types=[
        pltpu.VMEM((CH,), jnp.int32),
        pltpu.VMEM((CH,), jnp.int32),
        pltpu.VMEM((CH, D), jnp.float32),
        pltpu.VMEM((ZR, D), jnp.float32),
        pltpu.VMEM_SHARED((NUM_NODES, D), jnp.float32),
        pltpu.SemaphoreType.DMA,
    ],
)
def _aggregate_sum(h, src, dst, aggr_out, src_v, dst_v, bufm, zbuf, aggr_sh,
                   sem):
    cid = lax.axis_index("c")
    sid = lax.axis_index("s")
    base = (sid * NC + cid) * EW

    def zrow(r, carry):
        for k in range(D // 16):
            zbuf[r, pl.ds(k * 16, 16)] = jnp.zeros((16,), jnp.float32)
        return carry

    lax.fori_loop(0, ZR, zrow, 0, unroll=False)

    @pl.when(sid < DUMP_TILES)
    def _zero_shared():
        row0 = sid * ROWS_PER_TILE

        def zchunk(j, carry):
            pltpu.sync_copy(zbuf, aggr_sh.at[pl.ds(row0 + j * ZR, ZR)])
            return carry

        lax.fori_loop(0, NZCHUNK, zchunk, 0, unroll=False)

    plsc.subcore_barrier()

    def chunk(c, carry):
        off = base + c * CH
        pltpu.sync_copy(src.at[pl.ds(off, CH)], src_v)
        pltpu.sync_copy(dst.at[pl.ds(off, CH)], dst_v)
        pltpu.async_copy(h.at[src_v], bufm, sem).wait()
        pltpu.sync_copy(bufm, aggr_sh.at[dst_v], add=True)
        return carry

    lax.fori_loop(0, NCHUNK, chunk, 0, unroll=False)
    plsc.subcore_barrier()

    @pl.when(sid < DUMP_TILES)
    def _dump():
        row0 = sid * ROWS_PER_TILE

        def dchunk(j, carry):
            off = row0 + j * ZR
            pltpu.sync_copy(aggr_sh.at[pl.ds(off, ZR)], zbuf)
            pltpu.sync_copy(zbuf, aggr_out.at[cid, pl.ds(off, ZR)])
            return carry

        lax.fori_loop(0, NZCHUNK, dchunk, 0, unroll=False)


@functools.partial(
    pl.kernel,
    out_type=jax.ShapeDtypeStruct((NC, NUM_NODES, D), jnp.float32),
    mesh=_vec_mesh,
    scratch_types=[
        pltpu.VMEM((CH,), jnp.int32),
        pltpu.VMEM((CH, D), jnp.float32),
        pltpu.VMEM((ZR, D), jnp.float32),
        pltpu.VMEM_SHARED((NUM_NODES, D), jnp.float32),
    ],
)
def _aggregate_cnt(dst, cnt_out, dst_v, ones_v, zbuf2, cnt_sh):
    cid = lax.axis_index("c")
    sid = lax.axis_index("s")
    base = (sid * NC + cid) * EW

    def orow(r, carry):
        for k in range(D // 16):
            ones_v[r, pl.ds(k * 16, 16)] = jnp.ones((16,), jnp.float32)
        return carry

    lax.fori_loop(0, CH, orow, 0, unroll=False)

    def zrow2(r, carry):
        for k in range(D // 16):
            zbuf2[r, pl.ds(k * 16, 16)] = jnp.zeros((16,), jnp.float32)
        return carry

    lax.fori_loop(0, ZR, zrow2, 0, unroll=False)

    @pl.when(sid < DUMP_TILES)
    def _zero_shared():
        row0 = sid * ROWS_PER_TILE

        def zchunk(j, carry):
            pltpu.sync_copy(zbuf2, cnt_sh.at[pl.ds(row0 + j * ZR, ZR)])
            return carry

        lax.fori_loop(0, NZCHUNK, zchunk, 0, unroll=False)

    plsc.subcore_barrier()

    def chunk(c, carry):
        off = base + c * CH
        pltpu.sync_copy(dst.at[pl.ds(off, CH)], dst_v)
        pltpu.sync_copy(ones_v, cnt_sh.at[dst_v], add=True)
        return carry

    lax.fori_loop(0, NCHUNK, chunk, 0, unroll=False)
    plsc.subcore_barrier()

    @pl.when(sid < DUMP_TILES)
    def _dump():
        row0 = sid * ROWS_PER_TILE

        def dchunk(j, carry):
            off = row0 + j * ZR
            pltpu.sync_copy(cnt_sh.at[pl.ds(off, ZR)], zbuf2)
            pltpu.sync_copy(zbuf2, cnt_out.at[cid, pl.ds(off, ZR)])
            return carry

        lax.fori_loop(0, NZCHUNK, dchunk, 0, unroll=False)


def _sage_left_body(ap_ref, cp_ref, wl_ref, a_ref):
    aggr = ap_ref[0] + ap_ref[1]
    cnt = cp_ref[0, :, 0:1] + cp_ref[1, :, 0:1]
    aggr = aggr / jnp.maximum(cnt, 1.0)
    a_ref[...] = jnp.dot(aggr, wl_ref[...], preferred_element_type=jnp.float32)


def _sage_left(aggr_part, cnt_part, W_l):
    return pl.pallas_call(
        _sage_left_body,
        out_shape=jax.ShapeDtypeStruct((NUM_NODES, HID), jnp.float32),
    )(aggr_part, cnt_part, W_l)


BLK = 2000
NBLK = E // BLK
A_BLKS = NUM_NODES // BLK


def _mlp_body(h_ref, a_ref, wr_ref, bs_ref, w1_ref, b1_ref, w2_ref, b2_ref,
              w3_ref, b3_ref, out_ref):
    i = pl.program_id(0)
    z = jnp.dot(h_ref[...], wr_ref[...], preferred_element_type=jnp.float32)
    z = z + bs_ref[...]
    z = z + jnp.where(i < A_BLKS, a_ref[...], 0.0)
    z = jnp.maximum(z, 0.0)
    z = jnp.maximum(jnp.dot(z, w1_ref[...], preferred_element_type=jnp.float32)
                    + b1_ref[...], 0.0)
    z = jnp.maximum(jnp.dot(z, w2_ref[...], preferred_element_type=jnp.float32)
                    + b2_ref[...], 0.0)
    o = jnp.dot(z, w3_ref[...], preferred_element_type=jnp.float32) + b3_ref[...]
    m = jnp.max(o, axis=1, keepdims=True)
    lse = m + jnp.log(jnp.sum(jnp.exp(o - m), axis=1, keepdims=True))
    out_ref[...] = o - lse


def _mlp(h, a_small, W_r, b_s, W1, b1, W2, b2, W3, b3):
    full = lambda r, c: pl.BlockSpec((r, c), lambda i: (0, 0))
    return pl.pallas_call(
        _mlp_body,
        grid=(NBLK,),
        in_specs=[
            pl.BlockSpec((BLK, D), lambda i: (i, 0)),
            pl.BlockSpec((BLK, HID), lambda i: (jnp.minimum(i, A_BLKS - 1), 0)),
            full(D, HID), full(1, HID),
            full(HID, 64), full(1, 64),
            full(64, 32), full(1, 32),
            full(32, 2), full(1, 2),
        ],
        out_specs=pl.BlockSpec((BLK, 2), lambda i: (i, 0)),
        out_shape=jax.ShapeDtypeStruct((E, 2), jnp.float32),
    )(h, a_small, W_r, b_s, W1, b1, W2, b2, W3, b3)


def kernel(rna_f, protein_f, all_edges, W_sage_l, b_sage, W_sage_r,
           W1, b1, W2, b2, W3, b3):
    n_fea = jnp.concatenate([rna_f, protein_f], axis=0)
    pos = all_edges[::2]
    neg = jax.random.randint(jax.random.key(42), (2, E_POS), 0, NUM_NODES,
                             all_edges.dtype)
    src = jnp.concatenate([pos[:, 0], neg[0]])
    dst = jnp.concatenate([pos[:, 1], neg[1]])

    h = _edge_features(n_fea, src, dst)
    aggr_part = _aggregate_sum(h, src, dst)
    cnt_part = _aggregate_cnt(dst)
    a_small = _sage_left(aggr_part, cnt_part, W_sage_l)
    prob = _mlp(h, a_small, W_sage_r, b_sage.reshape(1, HID),
                W1, b1.reshape(1, 64), W2, b2.reshape(1, 32),
                W3, b3.reshape(1, 2))
    label = jnp.concatenate([jnp.ones((E_POS,), jnp.int32),
                             jnp.zeros((E_POS,), jnp.int32)])
    return (prob, label)


# speedup vs baseline: 4.0009x; 1.3991x over previous
import functools

import jax
import jax.numpy as jnp
from jax import lax
from jax.experimental import pallas as pl
from jax.experimental.pallas import tpu as pltpu
from jax.experimental.pallas import tpu_sc as plsc

N_RNA = 5000
N_PROT = 5000
NUM_NODES = N_RNA + N_PROT
D = 128
HID = 128
N_ALL_EDGES = 320000
E_POS = N_ALL_EDGES // 2
E = 2 * E_POS

NC = 2
NS = 16
NW = NC * NS
EW = E // NW
CH = 80
NCHUNK = EW // CH
NPAIR = (NCHUNK - 1) // 2

DUMP_TILES = 10
ROWS_PER_TILE = NUM_NODES // DUMP_TILES
ZR = 40
NZCHUNK = ROWS_PER_TILE // ZR

_vec_mesh = plsc.VectorSubcoreMesh(core_axis_name="c", subcore_axis_name="s")


@functools.partial(
    pl.kernel,
    out_type=jax.ShapeDtypeStruct((E, D), jnp.float32),
    mesh=_vec_mesh,
    scratch_types=[
        pltpu.VMEM((EW,), jnp.int32),
        pltpu.VMEM((EW,), jnp.int32),
        pltpu.VMEM((CH, D), jnp.float32),
        pltpu.VMEM((CH, D), jnp.float32),
        pltpu.VMEM((CH, D), jnp.float32),
        pltpu.VMEM((CH, D), jnp.float32),
        pltpu.VMEM((CH, D), jnp.float32),
        pltpu.VMEM((CH, D), jnp.float32),
        pltpu.SemaphoreType.DMA,
        pltpu.SemaphoreType.DMA,
        pltpu.SemaphoreType.DMA,
        pltpu.SemaphoreType.DMA,
    ],
)
def _edge_features(nfea, src, dst, h_out, src_v, dst_v,
                   bufa0, bufb0, bufh0, bufa1, bufb1, bufh1,
                   sa0, sb0, sa1, sb1):
    wid = lax.axis_index("s") * NC + lax.axis_index("c")
    base = wid * EW
    pltpu.sync_copy(src.at[pl.ds(base, EW)], src_v)
    pltpu.sync_copy(dst.at[pl.ds(base, EW)], dst_v)

    def start(c, ba, bb, s_a, s_b):
        off = c * CH
        pltpu.async_copy(nfea.at[src_v.at[pl.ds(off, CH)]], ba, s_a)
        pltpu.async_copy(nfea.at[dst_v.at[pl.ds(off, CH)]], bb, s_b)

    def wait(ba, bb, s_a, s_b):
        pltpu.make_async_copy(nfea.at[src_v.at[pl.ds(0, CH)]], ba, s_a).wait()
        pltpu.make_async_copy(nfea.at[dst_v.at[pl.ds(0, CH)]], bb, s_b).wait()

    def compute(c, ba, bb, bh):
        def row(r, carry2):
            for k in range(D // 16):
                sl = pl.ds(k * 16, 16)
                bh[r, sl] = ba[r, sl] * bb[r, sl]
            return carry2

        lax.fori_loop(0, CH, row, 0, unroll=False)
        pltpu.sync_copy(bh, h_out.at[pl.ds(base + c * CH, CH)])

    start(0, bufa0, bufb0, sa0, sb0)

    def pair(i, carry):
        c0 = 2 * i
        start(c0 + 1, bufa1, bufb1, sa1, sb1)
        wait(bufa0, bufb0, sa0, sb0)
        compute(c0, bufa0, bufb0, bufh0)
        start(c0 + 2, bufa0, bufb0, sa0, sb0)
        wait(bufa1, bufb1, sa1, sb1)
        compute(c0 + 1, bufa1, bufb1, bufh1)
        return carry

    lax.fori_loop(0, NPAIR, pair, 0, unroll=False)
    wait(bufa0, bufb0, sa0, sb0)
    compute(NCHUNK - 1, bufa0, bufb0, bufh0)


@functools.partial(
    pl.kernel,
    out_type=jax.ShapeDtypeStruct((NC, NUM_NODES, D), jnp.float32),
    mesh=_vec_mesh,
    scratch_types=[
        pltpu.VMEM((CH,), jnp.int32),
        pltpu.VMEM((CH,), jnp.int32),
        pltpu.VMEM((NCHUNK, CH), jnp.int32),
        pltpu.VMEM((CH, D), jnp.float32),
        pltpu.VMEM((CH, D), jnp.float32),
        pltpu.VMEM((ZR, D), jnp.float32),
        pltpu.VMEM_SHARED((NUM_NODES, D), jnp.float32),
        pltpu.SemaphoreType.DMA,
        pltpu.SemaphoreType.DMA,
    ],
)
def _aggregate_sum(h, src, dst3, aggr_out, sidx0, sidx1, dst_m, bufm0, bufm1,
                   zbuf, aggr_sh, sm0, sm1):
    cid = lax.axis_index("c")
    sid = lax.axis_index("s")
    wid = sid * NC + cid
    base = wid * EW
    pltpu.sync_copy(dst3.at[wid], dst_m)

    def zrow(r, carry):
        for k in range(D // 16):
            zbuf[r, pl.ds(k * 16, 16)] = jnp.zeros((16,), jnp.float32)
        return carry

    lax.fori_loop(0, ZR, zrow, 0, unroll=False)

    @pl.when(sid < DUMP_TILES)
    def _zero_shared():
        row0 = sid * ROWS_PER_TILE

        def zchunk(j, carry):
            pltpu.sync_copy(zbuf, aggr_sh.at[pl.ds(row0 + j * ZR, ZR)])
            return carry

        lax.fori_loop(0, NZCHUNK, zchunk, 0, unroll=False)

    plsc.subcore_barrier()

    def start(c, sidx, bm, s_m):
        pltpu.sync_copy(src.at[pl.ds(base + c * CH, CH)], sidx)
        pltpu.async_copy(h.at[sidx], bm, s_m)

    def wait(sidx, bm, s_m):
        pltpu.make_async_copy(h.at[sidx], bm, s_m).wait()

    def scatter(c, bm):
        pltpu.sync_copy(bm, aggr_sh.at[dst_m.at[c]], add=True)

    start(0, sidx0, bufm0, sm0)

    def pair(i, carry):
        c0 = 2 * i
        start(c0 + 1, sidx1, bufm1, sm1)
        wait(sidx0, bufm0, sm0)
        scatter(c0, bufm0)
        start(c0 + 2, sidx0, bufm0, sm0)
        wait(sidx1, bufm1, sm1)
        scatter(c0 + 1, bufm1)
        return carry

    lax.fori_loop(0, NPAIR, pair, 0, unroll=False)
    wait(sidx0, bufm0, sm0)
    scatter(NCHUNK - 1, bufm0)
    plsc.subcore_barrier()

    @pl.when(sid < DUMP_TILES)
    def _dump():
        row0 = sid * ROWS_PER_TILE

        def dchunk(j, carry):
            off = row0 + j * ZR
            pltpu.sync_copy(aggr_sh.at[pl.ds(off, ZR)], zbuf)
            pltpu.sync_copy(zbuf, aggr_out.at[cid, pl.ds(off, ZR)])
            return carry

        lax.fori_loop(0, NZCHUNK, dchunk, 0, unroll=False)


@functools.partial(
    pl.kernel,
    out_type=jax.ShapeDtypeStruct((NC, NUM_NODES, D), jnp.float32),
    mesh=_vec_mesh,
    scratch_types=[
        pltpu.VMEM((NCHUNK, CH), jnp.int32),
        pltpu.VMEM((CH, D), jnp.float32),
        pltpu.VMEM((ZR, D), jnp.float32),
        pltpu.VMEM_SHARED((NUM_NODES, D), jnp.float32),
    ],
)
def _aggregate_cnt(dst3, cnt_out, dst_m, ones_v, zbuf2, cnt_sh):
    cid = lax.axis_index("c")
    sid = lax.axis_index("s")
    wid = sid * NC + cid
    pltpu.sync_copy(dst3.at[wid], dst_m)

    def orow(r, carry):
        for k in range(D // 16):
            ones_v[r, pl.ds(k * 16, 16)] = jnp.ones((16,), jnp.float32)
        return carry

    lax.fori_loop(0, CH, orow, 0, unroll=False)

    def zrow2(r, carry):
        for k in range(D // 16):
            zbuf2[r, pl.ds(k * 16, 16)] = jnp.zeros((16,), jnp.float32)
        return carry

    lax.fori_loop(0, ZR, zrow2, 0, unroll=False)

    @pl.when(sid < DUMP_TILES)
    def _zero_shared():
        row0 = sid * ROWS_PER_TILE

        def zchunk(j, carry):
            pltpu.sync_copy(zbuf2, cnt_sh.at[pl.ds(row0 + j * ZR, ZR)])
            return carry

        lax.fori_loop(0, NZCHUNK, zchunk, 0, unroll=False)

    plsc.subcore_barrier()

    def chunk(c, carry):
        pltpu.sync_copy(ones_v, cnt_sh.at[dst_m.at[c]], add=True)
        return carry

    lax.fori_loop(0, NCHUNK, chunk, 0, unroll=False)
    plsc.subcore_barrier()

    @pl.when(sid < DUMP_TILES)
    def _dump():
        row0 = sid * ROWS_PER_TILE

        def dchunk(j, carry):
            off = row0 + j * ZR
            pltpu.sync_copy(cnt_sh.at[pl.ds(off, ZR)], zbuf2)
            pltpu.sync_copy(zbuf2, cnt_out.at[cid, pl.ds(off, ZR)])
            return carry

        lax.fori_loop(0, NZCHUNK, dchunk, 0, unroll=False)


def _sage_left_body(ap_ref, cp_ref, wl_ref, a_ref):
    aggr = ap_ref[0] + ap_ref[1]
    cnt = cp_ref[0, :, 0:1] + cp_ref[1, :, 0:1]
    aggr = aggr / jnp.maximum(cnt, 1.0)
    a_ref[...] = jnp.dot(aggr, wl_ref[...], preferred_element_type=jnp.float32)


def _sage_left(aggr_part, cnt_part, W_l):
    return pl.pallas_call(
        _sage_left_body,
        out_shape=jax.ShapeDtypeStruct((NUM_NODES, HID), jnp.float32),
    )(aggr_part, cnt_part, W_l)


BLK = 2000
NBLK = E // BLK
A_BLKS = NUM_NODES // BLK


def _mlp_body(h_ref, a_ref, wr_ref, bs_ref, w1_ref, b1_ref, w2_ref, b2_ref,
              w3_ref, b3_ref, out_ref):
    i = pl.program_id(0)
    z = jnp.dot(h_ref[...], wr_ref[...], preferred_element_type=jnp.float32)
    z = z + bs_ref[...]
    z = z + jnp.where(i < A_BLKS, a_ref[...], 0.0)
    z = jnp.maximum(z, 0.0)
    z = jnp.maximum(jnp.dot(z, w1_ref[...], preferred_element_type=jnp.float32)
                    + b1_ref[...], 0.0)
    z = jnp.maximum(jnp.dot(z, w2_ref[...], preferred_element_type=jnp.float32)
                    + b2_ref[...], 0.0)
    o = jnp.dot(z, w3_ref[...], preferred_element_type=jnp.float32) + b3_ref[...]
    m = jnp.max(o, axis=1, keepdims=True)
    lse = m + jnp.log(jnp.sum(jnp.exp(o - m), axis=1, keepdims=True))
    out_ref[...] = o - lse


def _mlp(h, a_small, W_r, b_s, W1, b1, W2, b2, W3, b3):
    full = lambda r, c: pl.BlockSpec((r, c), lambda i: (0, 0))
    return pl.pallas_call(
        _mlp_body,
        grid=(NBLK,),
        in_specs=[
            pl.BlockSpec((BLK, D), lambda i: (i, 0)),
            pl.BlockSpec((BLK, HID), lambda i: (jnp.minimum(i, A_BLKS - 1), 0)),
            full(D, HID), full(1, HID),
            full(HID, 64), full(1, 64),
            full(64, 32), full(1, 32),
            full(32, 2), full(1, 2),
        ],
        out_specs=pl.BlockSpec((BLK, 2), lambda i: (i, 0)),
        out_shape=jax.ShapeDtypeStruct((E, 2), jnp.float32),
    )(h, a_small, W_r, b_s, W1, b1, W2, b2, W3, b3)


def kernel(rna_f, protein_f, all_edges, W_sage_l, b_sage, W_sage_r,
           W1, b1, W2, b2, W3, b3):
    n_fea = jnp.concatenate([rna_f, protein_f], axis=0)
    pos = all_edges[::2]
    neg = jax.random.randint(jax.random.key(42), (2, E_POS), 0, NUM_NODES,
                             all_edges.dtype)
    src = jnp.concatenate([pos[:, 0], neg[0]])
    dst = jnp.concatenate([pos[:, 1], neg[1]])
    dst3 = dst.reshape(NW, NCHUNK, CH)

    h = _edge_features(n_fea, src, dst)
    aggr_part = _aggregate_sum(h, src, dst3)
    cnt_part = _aggregate_cnt(dst3)
    a_small = _sage_left(aggr_part, cnt_part, W_sage_l)
    prob = _mlp(h, a_small, W_sage_r, b_sage.reshape(1, HID),
                W1, b1.reshape(1, 64), W2, b2.reshape(1, 32),
                W3, b3.reshape(1, 2))
    label = jnp.concatenate([jnp.ones((E_POS,), jnp.int32),
                             jnp.zeros((E_POS,), jnp.int32)])
    return (prob, label)
